# baseline (speedup 1.0000x reference)
import functools

import jax
import jax.numpy as jnp
import numpy as np
from jax import lax
from jax.experimental import pallas as pl
from jax.experimental.pallas import tpu as pltpu
from jax.experimental.pallas import tpu_sc as plsc

EMBED_DIM = 1024
NUM_HEADS = 16
HEAD_DIM = 64
CODE_SIZE = 8
K_CODES = 256
SCALE = HEAD_DIM ** (-0.5)
B = 4
SEQ = 2048
L_TOTAL = B * SEQ

_code_base = 2 ** np.arange(CODE_SIZE - 1, -1, -1)
_bits = (np.arange(K_CODES)[:, None] // _code_base) % 2
_BASE_CODE = np.concatenate([_bits, 1 - _bits], axis=-1).astype(np.float32)

_HASH_BLK = 1024


def _hash_body(k_ref, w_ref, b_ref, seg_ref):
    blk = pl.program_id(0)
    s8 = lax.dot_general(w_ref[...], k_ref[...], (((1,), (1,)), ((), ())),
                         preferred_element_type=jnp.float32)
    s8 = s8 + b_ref[:, 0:1]
    bits = (s8 >= 0.0).astype(jnp.int32)
    j = lax.broadcasted_iota(jnp.int32, (CODE_SIZE, _HASH_BLK), 0)
    pw = jnp.left_shift(1, (CODE_SIZE - 1) - j)
    del blk
    idx = jnp.sum(bits * pw, axis=0, keepdims=True)
    seg_ref[...] = idx[None]


def _hash_call(k, w, b_bcast):
    nblk = L_TOTAL // _HASH_BLK
    seg3 = pl.pallas_call(
        _hash_body,
        grid=(nblk,),
        in_specs=[
            pl.BlockSpec((_HASH_BLK, EMBED_DIM), lambda i: (i, 0)),
            pl.BlockSpec((CODE_SIZE, EMBED_DIM), lambda i: (0, 0)),
            pl.BlockSpec((CODE_SIZE, 128), lambda i: (0, 0)),
        ],
        out_specs=pl.BlockSpec((1, 1, _HASH_BLK), lambda i: (i, 0, 0)),
        out_shape=jax.ShapeDtypeStruct((nblk, 1, _HASH_BLK), jnp.int32),
    )(k, w, b_bcast)
    return seg3.reshape(L_TOTAL)


_NC = 2
_NS = 16
_CHUNK = 64
_TOK_PER_TILE = L_TOTAL // (_NC * _NS)
_NCHUNK = _TOK_PER_TILE // _CHUNK
_SEGS = 2 * K_CODES
_ROWS_PER_TILE = _SEGS // _NS


_NCG = 8
_CW = EMBED_DIM // _NCG
_HTOK = SEQ
_CH = 128
_NCH = _HTOK // _CH


def _sc_body(v_hbm, seg_hbm, zv_hbm, zc_hbm, ones_hbm, cod_v_hbm, cnt_hbm,
             idx_all, vbuf, acc, cnt, one_v, sem0, sem1):
    c = lax.axis_index("c")
    s = lax.axis_index("s")
    cg = s % _NCG
    h = s // _NCG
    b = 2 * c + h
    row0 = b * SEQ
    col0 = cg * _CW

    pltpu.sync_copy(zv_hbm, acc)
    pltpu.sync_copy(zc_hbm, cnt)
    pltpu.sync_copy(ones_hbm, one_v)
    pltpu.sync_copy(seg_hbm.at[pl.ds(row0, _HTOK)], idx_all)
    ov = one_v[...]

    sems = (sem0, sem1)

    pltpu.async_copy(v_hbm.at[pl.ds(row0, _CH), pl.ds(col0, _CW)],
                     vbuf.at[0], sems[0])

    def _chunk_pair(jj, _):
        for bsel in range(2):
            j = jj * 2 + bsel
            nxt = ((j + 1) % _NCH) * _CH
            pltpu.async_copy(
                v_hbm.at[pl.ds(row0 + nxt, _CH), pl.ds(col0, _CW)],
                vbuf.at[1 - bsel], sems[1 - bsel])
            pltpu.make_async_copy(
                v_hbm.at[pl.ds(row0, _CH), pl.ds(col0, _CW)],
                vbuf.at[bsel], sems[bsel]).wait()
            vb = vbuf.at[bsel]

            @plsc.parallel_loop(0, _CH // 16, unroll=2)
            def _grp(g):
                segv = idx_all[pl.ds(j * _CH + g * 16, 16)]
                for i in range(16):
                    sr = segv[i]
                    r = g * 16 + i
                    for q in range(_CW // 16):
                        plsc.addupdate(acc.at[sr, pl.ds(q * 16, 16)],
                                       vb[r, pl.ds(q * 16, 16)])
        return 0

    lax.fori_loop(0, _NCH // 2, _chunk_pair, 0)
    pltpu.make_async_copy(v_hbm.at[pl.ds(row0, _CH), pl.ds(col0, _CW)],
                          vbuf.at[0], sems[0]).wait()

    @pl.when(cg == 0)
    def _():
        def _cgrp(g, _):
            segv = idx_all[pl.ds(g * 16, 16)]
            plsc.addupdate_scatter(cnt, [segv], ov)
            return 0

        lax.fori_loop(0, _HTOK // 16, _cgrp, 0)
        pltpu.sync_copy(cnt, cnt_hbm.at[b])

    pltpu.sync_copy(acc, cod_v_hbm.at[pl.ds(b * K_CODES, K_CODES),
                                      pl.ds(col0, _CW)])


@functools.cache
def _sc_scatter_fn():
    return functools.partial(
        pl.kernel,
        out_type=[jax.ShapeDtypeStruct((B * K_CODES, EMBED_DIM), jnp.float32),
                  jax.ShapeDtypeStruct((B, K_CODES), jnp.float32)],
        mesh=plsc.VectorSubcoreMesh(core_axis_name="c", subcore_axis_name="s",
                                    num_cores=_NC, num_subcores=_NS),
        compiler_params=pltpu.CompilerParams(use_tc_tiling_on_sc=True,
                                             needs_layout_passes=False),
        scratch_types=[
            pltpu.VMEM((_HTOK,), jnp.int32),
            pltpu.VMEM((2, _CH, _CW), jnp.float32),
            pltpu.VMEM((K_CODES, _CW), jnp.float32),
            pltpu.VMEM((K_CODES,), jnp.float32),
            pltpu.VMEM((16,), jnp.float32),
            pltpu.SemaphoreType.DMA,
            pltpu.SemaphoreType.DMA,
        ],
    )(_sc_body)


_BQ = 512
_NQ = SEQ // _BQ
_NP = NUM_HEADS // 2


def _attn_body(q_ref, cb_ref, bt_ref, v_ref, c_ref, o_ref):
    cvec = c_ref[0, 0, :]
    pos = cvec > 0.0
    safe = jnp.maximum(cvec, 1.0)
    bias = jnp.where(pos, jnp.log(safe), -1e30)
    inv = jnp.where(pos, 1.0 / safe, 0.0)
    cbf_t = lax.dot_general(cb_ref[...], bt_ref[...], (((0,), (0,)), ((), ())),
                            preferred_element_type=jnp.float32)
    z64 = jnp.zeros((HEAD_DIM, K_CODES), jnp.float32)
    k2 = jnp.concatenate(
        [jnp.concatenate([cbf_t[:HEAD_DIM], z64], axis=1),
         jnp.concatenate([z64, cbf_t[HEAD_DIM:]], axis=1)], axis=0)
    s = lax.dot_general(q_ref[...], k2, (((1,), (0,)), ((), ())),
                        preferred_element_type=jnp.float32)
    s = s * SCALE + jnp.concatenate([bias, bias])[None, :]
    sa = s[:, :K_CODES]
    sb = s[:, K_CODES:]
    pa = jnp.exp(sa - jnp.max(sa, axis=1, keepdims=True))
    pb = jnp.exp(sb - jnp.max(sb, axis=1, keepdims=True))
    wa = pa * (inv[None, :] / jnp.sum(pa, axis=1, keepdims=True))
    wb = pb * (inv[None, :] / jnp.sum(pb, axis=1, keepdims=True))
    na = lax.dot_general(wa, v_ref[:, :HEAD_DIM], (((1,), (0,)), ((), ())),
                         preferred_element_type=jnp.float32)
    nb = lax.dot_general(wb, v_ref[:, HEAD_DIM:], (((1,), (0,)), ((), ())),
                         preferred_element_type=jnp.float32)
    o_ref[...] = jnp.concatenate([na, nb], axis=1)


def _attn_call(q, codebook, base_t, cod_v, counts3):
    return pl.pallas_call(
        _attn_body,
        grid=(B, _NP, _NQ),
        in_specs=[
            pl.BlockSpec((_BQ, 2 * HEAD_DIM), lambda b, p, j: (b * _NQ + j, p)),
            pl.BlockSpec((2 * CODE_SIZE, 2 * HEAD_DIM), lambda b, p, j: (0, p)),
            pl.BlockSpec((2 * CODE_SIZE, K_CODES), lambda b, p, j: (0, 0)),
            pl.BlockSpec((K_CODES, 2 * HEAD_DIM), lambda b, p, j: (b, p)),
            pl.BlockSpec((1, 1, K_CODES), lambda b, p, j: (b, 0, 0)),
        ],
        out_specs=pl.BlockSpec((_BQ, 2 * HEAD_DIM), lambda b, p, j: (b * _NQ + j, p)),
        out_shape=jax.ShapeDtypeStruct((L_TOTAL, EMBED_DIM), jnp.float32),
    )(q, codebook, base_t, cod_v, counts3)


def kernel(q, k, v, code_proj_w, code_proj_b, codebook, lengths, inv_lengths):
    del lengths, inv_lengths
    b_bcast = jnp.broadcast_to(code_proj_b[:, None], (CODE_SIZE, 128))
    seg = _hash_call(k, code_proj_w, b_bcast)
    zv = jnp.zeros((K_CODES, _CW), jnp.float32)
    zc = jnp.zeros((K_CODES,), jnp.float32)
    ones = jnp.ones((16,), jnp.float32)
    cod_v, cnt = _sc_scatter_fn()(v, seg, zv, zc, ones)
    counts3 = cnt.reshape(B, 1, K_CODES)
    return _attn_call(q, codebook, jnp.asarray(_BASE_CODE.T), cod_v, counts3)


# speedup vs baseline: 1.0195x; 1.0195x over previous
import functools

import jax
import jax.numpy as jnp
import numpy as np
from jax import lax
from jax.experimental import pallas as pl
from jax.experimental.pallas import tpu as pltpu
from jax.experimental.pallas import tpu_sc as plsc

EMBED_DIM = 1024
NUM_HEADS = 16
HEAD_DIM = 64
CODE_SIZE = 8
K_CODES = 256
SCALE = HEAD_DIM ** (-0.5)
B = 4
SEQ = 2048
L_TOTAL = B * SEQ

_code_base = 2 ** np.arange(CODE_SIZE - 1, -1, -1)
_bits = (np.arange(K_CODES)[:, None] // _code_base) % 2
_BASE_CODE = np.concatenate([_bits, 1 - _bits], axis=-1).astype(np.float32)

_HASH_BLK = 1024


def _hash_body(k_ref, w_ref, b_ref, seg_ref):
    blk = pl.program_id(0)
    s8 = lax.dot_general(w_ref[...], k_ref[...], (((1,), (1,)), ((), ())),
                         preferred_element_type=jnp.float32)
    s8 = s8 + b_ref[:, 0:1]
    bits = (s8 >= 0.0).astype(jnp.int32)
    j = lax.broadcasted_iota(jnp.int32, (CODE_SIZE, _HASH_BLK), 0)
    pw = jnp.left_shift(1, (CODE_SIZE - 1) - j)
    del blk
    idx = jnp.sum(bits * pw, axis=0, keepdims=True)
    seg_ref[...] = idx[None]


def _hash_call(k, w, b_bcast):
    nblk = L_TOTAL // _HASH_BLK
    seg3 = pl.pallas_call(
        _hash_body,
        grid=(nblk,),
        in_specs=[
            pl.BlockSpec((_HASH_BLK, EMBED_DIM), lambda i: (i, 0)),
            pl.BlockSpec((CODE_SIZE, EMBED_DIM), lambda i: (0, 0)),
            pl.BlockSpec((CODE_SIZE, 128), lambda i: (0, 0)),
        ],
        out_specs=pl.BlockSpec((1, 1, _HASH_BLK), lambda i: (i, 0, 0)),
        out_shape=jax.ShapeDtypeStruct((nblk, 1, _HASH_BLK), jnp.int32),
    )(k, w, b_bcast)
    return seg3.reshape(L_TOTAL)


_NC = 2
_NS = 16
_CHUNK = 64
_TOK_PER_TILE = L_TOTAL // (_NC * _NS)
_NCHUNK = _TOK_PER_TILE // _CHUNK
_SEGS = 2 * K_CODES
_ROWS_PER_TILE = _SEGS // _NS


_NCG = 8
_CW = EMBED_DIM // _NCG
_HTOK = SEQ
_CH = 128
_NCH = _HTOK // _CH


def _sc_body(v_hbm, seg_hbm, zv_hbm, zc_hbm, ones_hbm, cod_v_hbm, cnt_hbm,
             idx_all, vbuf, acc, cnt, one_v, sem0, sem1):
    c = lax.axis_index("c")
    s = lax.axis_index("s")
    cg = s % _NCG
    h = s // _NCG
    b = 2 * c + h
    row0 = b * SEQ
    col0 = cg * _CW

    pltpu.sync_copy(zv_hbm, acc)
    pltpu.sync_copy(zc_hbm, cnt)
    pltpu.sync_copy(ones_hbm, one_v)
    pltpu.sync_copy(seg_hbm.at[pl.ds(row0, _HTOK)], idx_all)
    ov = one_v[...]

    sems = (sem0, sem1)

    pltpu.async_copy(v_hbm.at[pl.ds(row0, _CH), pl.ds(col0, _CW)],
                     vbuf.at[0], sems[0])

    def _chunk_pair(jj, _):
        for bsel in range(2):
            j = jj * 2 + bsel
            nxt = ((j + 1) % _NCH) * _CH
            pltpu.async_copy(
                v_hbm.at[pl.ds(row0 + nxt, _CH), pl.ds(col0, _CW)],
                vbuf.at[1 - bsel], sems[1 - bsel])
            pltpu.make_async_copy(
                v_hbm.at[pl.ds(row0, _CH), pl.ds(col0, _CW)],
                vbuf.at[bsel], sems[bsel]).wait()
            vb = vbuf.at[bsel]

            @plsc.parallel_loop(0, _CH // 16, unroll=2)
            def _grp(g):
                segv = idx_all[pl.ds(j * _CH + g * 16, 16)]
                for i in range(16):
                    sr = segv[i]
                    r = g * 16 + i
                    for q in range(_CW // 16):
                        plsc.addupdate(acc.at[sr, pl.ds(q * 16, 16)],
                                       vb[r, pl.ds(q * 16, 16)])
        return 0

    lax.fori_loop(0, _NCH // 2, _chunk_pair, 0)
    pltpu.make_async_copy(v_hbm.at[pl.ds(row0, _CH), pl.ds(col0, _CW)],
                          vbuf.at[0], sems[0]).wait()

    @pl.when(cg == 0)
    def _():
        def _cgrp(g, _):
            segv = idx_all[pl.ds(g * 16, 16)]
            plsc.addupdate_scatter(cnt, [segv], ov)
            return 0

        lax.fori_loop(0, _HTOK // 16, _cgrp, 0)
        pltpu.sync_copy(cnt, cnt_hbm.at[b])

    pltpu.sync_copy(acc, cod_v_hbm.at[pl.ds(b * K_CODES, K_CODES),
                                      pl.ds(col0, _CW)])


@functools.cache
def _sc_scatter_fn():
    return functools.partial(
        pl.kernel,
        out_type=[jax.ShapeDtypeStruct((B * K_CODES, EMBED_DIM), jnp.float32),
                  jax.ShapeDtypeStruct((B, K_CODES), jnp.float32)],
        mesh=plsc.VectorSubcoreMesh(core_axis_name="c", subcore_axis_name="s",
                                    num_cores=_NC, num_subcores=_NS),
        compiler_params=pltpu.CompilerParams(use_tc_tiling_on_sc=True,
                                             needs_layout_passes=False),
        scratch_types=[
            pltpu.VMEM((_HTOK,), jnp.int32),
            pltpu.VMEM((2, _CH, _CW), jnp.float32),
            pltpu.VMEM((K_CODES, _CW), jnp.float32),
            pltpu.VMEM((K_CODES,), jnp.float32),
            pltpu.VMEM((16,), jnp.float32),
            pltpu.SemaphoreType.DMA,
            pltpu.SemaphoreType.DMA,
        ],
    )(_sc_body)


_BQ = 512
_NQ = SEQ // _BQ
_NP = NUM_HEADS // 2


def _attn_body(q_ref, cb_ref, bt_ref, v_ref, c_ref, o_ref, k2_ref):
    cvec = c_ref[0, 0, :]
    pos = cvec > 0.0
    safe = jnp.maximum(cvec, 1.0)
    bias = jnp.where(pos, jnp.log(safe), -1e30)
    inv = jnp.where(pos, 1.0 / safe, 0.0)

    @pl.when(pl.program_id(2) == 0)
    def _():
        cbf_t = lax.dot_general(cb_ref[...], bt_ref[...],
                                (((0,), (0,)), ((), ())),
                                preferred_element_type=jnp.float32)
        z64 = jnp.zeros((HEAD_DIM, K_CODES), jnp.float32)
        k2_ref[...] = jnp.concatenate(
            [jnp.concatenate([cbf_t[:HEAD_DIM], z64], axis=1),
             jnp.concatenate([z64, cbf_t[HEAD_DIM:]], axis=1)], axis=0)

    s = lax.dot_general(q_ref[...], k2_ref[...], (((1,), (0,)), ((), ())),
                        preferred_element_type=jnp.float32)
    s = s * SCALE + jnp.concatenate([bias, bias])[None, :]
    sa = s[:, :K_CODES]
    sb = s[:, K_CODES:]
    pa = jnp.exp(sa - jnp.max(sa, axis=1, keepdims=True))
    pb = jnp.exp(sb - jnp.max(sb, axis=1, keepdims=True))
    wa = pa * (inv[None, :] / jnp.sum(pa, axis=1, keepdims=True))
    wb = pb * (inv[None, :] / jnp.sum(pb, axis=1, keepdims=True))
    na = lax.dot_general(wa, v_ref[:, :HEAD_DIM], (((1,), (0,)), ((), ())),
                         preferred_element_type=jnp.float32)
    nb = lax.dot_general(wb, v_ref[:, HEAD_DIM:], (((1,), (0,)), ((), ())),
                         preferred_element_type=jnp.float32)
    o_ref[...] = jnp.concatenate([na, nb], axis=1)


def _attn_call(q, codebook, base_t, cod_v, counts3):
    return pl.pallas_call(
        _attn_body,
        grid=(B, _NP, _NQ),
        in_specs=[
            pl.BlockSpec((_BQ, 2 * HEAD_DIM), lambda b, p, j: (b * _NQ + j, p)),
            pl.BlockSpec((2 * CODE_SIZE, 2 * HEAD_DIM), lambda b, p, j: (0, p)),
            pl.BlockSpec((2 * CODE_SIZE, K_CODES), lambda b, p, j: (0, 0)),
            pl.BlockSpec((K_CODES, 2 * HEAD_DIM), lambda b, p, j: (b, p)),
            pl.BlockSpec((1, 1, K_CODES), lambda b, p, j: (b, 0, 0)),
        ],
        out_specs=pl.BlockSpec((_BQ, 2 * HEAD_DIM), lambda b, p, j: (b * _NQ + j, p)),
        out_shape=jax.ShapeDtypeStruct((L_TOTAL, EMBED_DIM), jnp.float32),
        scratch_shapes=[pltpu.VMEM((2 * HEAD_DIM, 2 * K_CODES), jnp.float32)],
    )(q, codebook, base_t, cod_v, counts3)


def kernel(q, k, v, code_proj_w, code_proj_b, codebook, lengths, inv_lengths):
    del lengths, inv_lengths
    b_bcast = jnp.broadcast_to(code_proj_b[:, None], (CODE_SIZE, 128))
    seg = _hash_call(k, code_proj_w, b_bcast)
    zv = jnp.zeros((K_CODES, _CW), jnp.float32)
    zc = jnp.zeros((K_CODES,), jnp.float32)
    ones = jnp.ones((16,), jnp.float32)
    cod_v, cnt = _sc_scatter_fn()(v, seg, zv, zc, ones)
    counts3 = cnt.reshape(B, 1, K_CODES)
    return _attn_call(q, codebook, jnp.asarray(_BASE_CODE.T), cod_v, counts3)


# speedup vs baseline: 1.0504x; 1.0303x over previous
import functools

import jax
import jax.numpy as jnp
import numpy as np
from jax import lax
from jax.experimental import pallas as pl
from jax.experimental.pallas import tpu as pltpu
from jax.experimental.pallas import tpu_sc as plsc

EMBED_DIM = 1024
NUM_HEADS = 16
HEAD_DIM = 64
CODE_SIZE = 8
K_CODES = 256
SCALE = HEAD_DIM ** (-0.5)
B = 4
SEQ = 2048
L_TOTAL = B * SEQ

_code_base = 2 ** np.arange(CODE_SIZE - 1, -1, -1)
_bits = (np.arange(K_CODES)[:, None] // _code_base) % 2
_BASE_CODE = np.concatenate([_bits, 1 - _bits], axis=-1).astype(np.float32)

_HASH_BLK = 1024


def _hash_body(k_ref, w_ref, b_ref, seg_ref):
    blk = pl.program_id(0)
    s8 = lax.dot_general(w_ref[...], k_ref[...], (((1,), (1,)), ((), ())),
                         preferred_element_type=jnp.float32)
    s8 = s8 + b_ref[:, 0:1]
    bits = (s8 >= 0.0).astype(jnp.int32)
    j = lax.broadcasted_iota(jnp.int32, (CODE_SIZE, _HASH_BLK), 0)
    pw = jnp.left_shift(1, (CODE_SIZE - 1) - j)
    del blk
    idx = jnp.sum(bits * pw, axis=0, keepdims=True)
    seg_ref[...] = idx[None]


def _hash_call(k, w, b_bcast):
    nblk = L_TOTAL // _HASH_BLK
    seg3 = pl.pallas_call(
        _hash_body,
        grid=(nblk,),
        in_specs=[
            pl.BlockSpec((_HASH_BLK, EMBED_DIM), lambda i: (i, 0)),
            pl.BlockSpec((CODE_SIZE, EMBED_DIM), lambda i: (0, 0)),
            pl.BlockSpec((CODE_SIZE, 128), lambda i: (0, 0)),
        ],
        out_specs=pl.BlockSpec((1, 1, _HASH_BLK), lambda i: (i, 0, 0)),
        out_shape=jax.ShapeDtypeStruct((nblk, 1, _HASH_BLK), jnp.int32),
    )(k, w, b_bcast)
    return seg3.reshape(L_TOTAL)


_NC = 2
_NS = 16
_CHUNK = 64
_TOK_PER_TILE = L_TOTAL // (_NC * _NS)
_NCHUNK = _TOK_PER_TILE // _CHUNK
_SEGS = 2 * K_CODES
_ROWS_PER_TILE = _SEGS // _NS


_NCG = 8
_CW = EMBED_DIM // _NCG
_HTOK = SEQ
_CH = 128
_NCH = _HTOK // _CH


def _sc_body(v_hbm, seg_hbm, zv_hbm, zc_hbm, ones_hbm, cod_v_hbm, cnt_hbm,
             idx_all, vbuf, acc, cnt, one_v, sem0, sem1):
    c = lax.axis_index("c")
    s = lax.axis_index("s")
    cg = s % _NCG
    h = s // _NCG
    b = 2 * c + h
    row0 = b * SEQ
    col0 = cg * _CW

    pltpu.sync_copy(zv_hbm, acc)
    pltpu.sync_copy(zc_hbm, cnt)
    pltpu.sync_copy(ones_hbm, one_v)
    pltpu.sync_copy(seg_hbm.at[pl.ds(row0, _HTOK)], idx_all)
    ov = one_v[...]

    sems = (sem0, sem1)

    pltpu.async_copy(v_hbm.at[pl.ds(row0, _CH), pl.ds(col0, _CW)],
                     vbuf.at[0], sems[0])

    def _chunk_pair(jj, _):
        for bsel in range(2):
            j = jj * 2 + bsel
            nxt = ((j + 1) % _NCH) * _CH
            pltpu.async_copy(
                v_hbm.at[pl.ds(row0 + nxt, _CH), pl.ds(col0, _CW)],
                vbuf.at[1 - bsel], sems[1 - bsel])
            pltpu.make_async_copy(
                v_hbm.at[pl.ds(row0, _CH), pl.ds(col0, _CW)],
                vbuf.at[bsel], sems[bsel]).wait()
            vb = vbuf.at[bsel]

            @plsc.parallel_loop(0, _CH // 16)
            def _grp(g):
                segv = idx_all[pl.ds(j * _CH + g * 16, 16)]
                for i in range(16):
                    sr = segv[i]
                    r = g * 16 + i
                    for q in range(_CW // 16):
                        plsc.addupdate(acc.at[sr, pl.ds(q * 16, 16)],
                                       vb[r, pl.ds(q * 16, 16)])
        return 0

    lax.fori_loop(0, _NCH // 2, _chunk_pair, 0)
    pltpu.make_async_copy(v_hbm.at[pl.ds(row0, _CH), pl.ds(col0, _CW)],
                          vbuf.at[0], sems[0]).wait()

    @pl.when(cg == 0)
    def _():
        def _cgrp(g, _):
            segv = idx_all[pl.ds(g * 16, 16)]
            plsc.addupdate_scatter(cnt, [segv], ov)
            return 0

        lax.fori_loop(0, _HTOK // 16, _cgrp, 0)
        pltpu.sync_copy(cnt, cnt_hbm.at[b])

    pltpu.sync_copy(acc, cod_v_hbm.at[pl.ds(b * K_CODES, K_CODES),
                                      pl.ds(col0, _CW)])


@functools.cache
def _sc_scatter_fn():
    return functools.partial(
        pl.kernel,
        out_type=[jax.ShapeDtypeStruct((B * K_CODES, EMBED_DIM), jnp.float32),
                  jax.ShapeDtypeStruct((B, K_CODES), jnp.float32)],
        mesh=plsc.VectorSubcoreMesh(core_axis_name="c", subcore_axis_name="s",
                                    num_cores=_NC, num_subcores=_NS),
        compiler_params=pltpu.CompilerParams(use_tc_tiling_on_sc=True,
                                             needs_layout_passes=False),
        scratch_types=[
            pltpu.VMEM((_HTOK,), jnp.int32),
            pltpu.VMEM((2, _CH, _CW), jnp.float32),
            pltpu.VMEM((K_CODES, _CW), jnp.float32),
            pltpu.VMEM((K_CODES,), jnp.float32),
            pltpu.VMEM((16,), jnp.float32),
            pltpu.SemaphoreType.DMA,
            pltpu.SemaphoreType.DMA,
        ],
    )(_sc_body)


_BQ = 512
_NQ = SEQ // _BQ
_NP = NUM_HEADS // 2


def _attn_body(q_ref, cb_ref, bt_ref, v_ref, c_ref, o_ref, k2_ref):
    cvec = c_ref[0, 0, :]
    pos = cvec > 0.0
    safe = jnp.maximum(cvec, 1.0)
    bias = jnp.where(pos, jnp.log(safe), -1e30)
    inv = jnp.where(pos, 1.0 / safe, 0.0)

    @pl.when(pl.program_id(2) == 0)
    def _():
        cbf_t = lax.dot_general(cb_ref[...], bt_ref[...],
                                (((0,), (0,)), ((), ())),
                                preferred_element_type=jnp.float32)
        z64 = jnp.zeros((HEAD_DIM, K_CODES), jnp.float32)
        k2_ref[...] = jnp.concatenate(
            [jnp.concatenate([cbf_t[:HEAD_DIM], z64], axis=1),
             jnp.concatenate([z64, cbf_t[HEAD_DIM:]], axis=1)], axis=0)

    s = lax.dot_general(q_ref[...], k2_ref[...], (((1,), (0,)), ((), ())),
                        preferred_element_type=jnp.float32)
    s = s * SCALE + jnp.concatenate([bias, bias])[None, :]
    sa = s[:, :K_CODES]
    sb = s[:, K_CODES:]
    pa = jnp.exp(sa - jnp.max(sa, axis=1, keepdims=True))
    pb = jnp.exp(sb - jnp.max(sb, axis=1, keepdims=True))
    wa = pa * (inv[None, :] / jnp.sum(pa, axis=1, keepdims=True))
    wb = pb * (inv[None, :] / jnp.sum(pb, axis=1, keepdims=True))
    na = lax.dot_general(wa, v_ref[:, :HEAD_DIM], (((1,), (0,)), ((), ())),
                         preferred_element_type=jnp.float32)
    nb = lax.dot_general(wb, v_ref[:, HEAD_DIM:], (((1,), (0,)), ((), ())),
                         preferred_element_type=jnp.float32)
    o_ref[...] = jnp.concatenate([na, nb], axis=1)


def _attn_call(q, codebook, base_t, cod_v, counts3):
    return pl.pallas_call(
        _attn_body,
        grid=(B, _NP, _NQ),
        in_specs=[
            pl.BlockSpec((_BQ, 2 * HEAD_DIM), lambda b, p, j: (b * _NQ + j, p)),
            pl.BlockSpec((2 * CODE_SIZE, 2 * HEAD_DIM), lambda b, p, j: (0, p)),
            pl.BlockSpec((2 * CODE_SIZE, K_CODES), lambda b, p, j: (0, 0)),
            pl.BlockSpec((K_CODES, 2 * HEAD_DIM), lambda b, p, j: (b, p)),
            pl.BlockSpec((1, 1, K_CODES), lambda b, p, j: (b, 0, 0)),
        ],
        out_specs=pl.BlockSpec((_BQ, 2 * HEAD_DIM), lambda b, p, j: (b * _NQ + j, p)),
        out_shape=jax.ShapeDtypeStruct((L_TOTAL, EMBED_DIM), jnp.float32),
        scratch_shapes=[pltpu.VMEM((2 * HEAD_DIM, 2 * K_CODES), jnp.float32)],
    )(q, codebook, base_t, cod_v, counts3)


def kernel(q, k, v, code_proj_w, code_proj_b, codebook, lengths, inv_lengths):
    del lengths, inv_lengths
    b_bcast = jnp.broadcast_to(code_proj_b[:, None], (CODE_SIZE, 128))
    seg = _hash_call(k, code_proj_w, b_bcast)
    zv = jnp.zeros((K_CODES, _CW), jnp.float32)
    zc = jnp.zeros((K_CODES,), jnp.float32)
    ones = jnp.ones((16,), jnp.float32)
    cod_v, cnt = _sc_scatter_fn()(v, seg, zv, zc, ones)
    counts3 = cnt.reshape(B, 1, K_CODES)
    return _attn_call(q, codebook, jnp.asarray(_BASE_CODE.T), cod_v, counts3)


# speedup vs baseline: 1.0837x; 1.0317x over previous
import functools

import jax
import jax.numpy as jnp
import numpy as np
from jax import lax
from jax.experimental import pallas as pl
from jax.experimental.pallas import tpu as pltpu
from jax.experimental.pallas import tpu_sc as plsc

EMBED_DIM = 1024
NUM_HEADS = 16
HEAD_DIM = 64
CODE_SIZE = 8
K_CODES = 256
SCALE = HEAD_DIM ** (-0.5)
B = 4
SEQ = 2048
L_TOTAL = B * SEQ

_code_base = 2 ** np.arange(CODE_SIZE - 1, -1, -1)
_bits = (np.arange(K_CODES)[:, None] // _code_base) % 2
_BASE_CODE = np.concatenate([_bits, 1 - _bits], axis=-1).astype(np.float32)

_HASH_BLK = 1024


def _hash_body(k_ref, w_ref, b_ref, seg_ref):
    blk = pl.program_id(0)
    s8 = lax.dot_general(w_ref[...], k_ref[...], (((1,), (1,)), ((), ())),
                         preferred_element_type=jnp.float32)
    s8 = s8 + b_ref[:, 0:1]
    bits = (s8 >= 0.0).astype(jnp.int32)
    j = lax.broadcasted_iota(jnp.int32, (CODE_SIZE, _HASH_BLK), 0)
    pw = jnp.left_shift(1, (CODE_SIZE - 1) - j)
    del blk
    idx = jnp.sum(bits * pw, axis=0, keepdims=True)
    seg_ref[...] = idx[None]


def _hash_call(k, w, b_bcast):
    nblk = L_TOTAL // _HASH_BLK
    seg3 = pl.pallas_call(
        _hash_body,
        grid=(nblk,),
        in_specs=[
            pl.BlockSpec((_HASH_BLK, EMBED_DIM), lambda i: (i, 0)),
            pl.BlockSpec((CODE_SIZE, EMBED_DIM), lambda i: (0, 0)),
            pl.BlockSpec((CODE_SIZE, 128), lambda i: (0, 0)),
        ],
        out_specs=pl.BlockSpec((1, 1, _HASH_BLK), lambda i: (i, 0, 0)),
        out_shape=jax.ShapeDtypeStruct((nblk, 1, _HASH_BLK), jnp.int32),
    )(k, w, b_bcast)
    return seg3.reshape(L_TOTAL)


_NC = 2
_NS = 16
_CHUNK = 64
_TOK_PER_TILE = L_TOTAL // (_NC * _NS)
_NCHUNK = _TOK_PER_TILE // _CHUNK
_SEGS = 2 * K_CODES
_ROWS_PER_TILE = _SEGS // _NS


_NCG = 8
_CW = EMBED_DIM // _NCG
_HTOK = SEQ
_CH = 128
_NCH = _HTOK // _CH


def _sc_body(v_hbm, seg_hbm, zv_hbm, zc_hbm, ones_hbm, cod_v_hbm, cnt_hbm,
             idx_all, vbuf, acc, cnt, one_v, sem0, sem1):
    c = lax.axis_index("c")
    s = lax.axis_index("s")
    cg = s % _NCG
    h = s // _NCG
    b = 2 * c + h
    row0 = b * SEQ
    col0 = cg * _CW

    pltpu.sync_copy(zv_hbm, acc)
    pltpu.sync_copy(zc_hbm, cnt)
    pltpu.sync_copy(ones_hbm, one_v)
    pltpu.sync_copy(seg_hbm.at[pl.ds(row0, _HTOK)], idx_all)
    ov = one_v[...]

    sems = (sem0, sem1)

    pltpu.async_copy(v_hbm.at[pl.ds(row0, _CH), pl.ds(col0, _CW)],
                     vbuf.at[0], sems[0])

    def _chunk_pair(jj, _):
        for bsel in range(2):
            j = jj * 2 + bsel
            nxt = ((j + 1) % _NCH) * _CH
            pltpu.async_copy(
                v_hbm.at[pl.ds(row0 + nxt, _CH), pl.ds(col0, _CW)],
                vbuf.at[1 - bsel], sems[1 - bsel])
            pltpu.make_async_copy(
                v_hbm.at[pl.ds(row0, _CH), pl.ds(col0, _CW)],
                vbuf.at[bsel], sems[bsel]).wait()
            vb = vbuf.at[bsel]

            @plsc.parallel_loop(0, _CH // 16)
            def _grp(g):
                segv = idx_all[pl.ds(j * _CH + g * 16, 16)]
                for i in range(16):
                    sr = segv[i]
                    r = g * 16 + i
                    for q in range(_CW // 16):
                        plsc.addupdate(acc.at[sr, pl.ds(q * 16, 16)],
                                       vb[r, pl.ds(q * 16, 16)])
        return 0

    lax.fori_loop(0, _NCH // 2, _chunk_pair, 0)
    pltpu.make_async_copy(v_hbm.at[pl.ds(row0, _CH), pl.ds(col0, _CW)],
                          vbuf.at[0], sems[0]).wait()

    @pl.when(cg == 0)
    def _():
        def _cgrp(g, _):
            segv = idx_all[pl.ds(g * 16, 16)]
            plsc.addupdate_scatter(cnt, [segv], ov)
            return 0

        lax.fori_loop(0, _HTOK // 16, _cgrp, 0)
        pltpu.sync_copy(cnt, cnt_hbm.at[b])

    pltpu.sync_copy(acc, cod_v_hbm.at[pl.ds(b * K_CODES, K_CODES),
                                      pl.ds(col0, _CW)])


@functools.cache
def _sc_scatter_fn():
    return functools.partial(
        pl.kernel,
        out_type=[jax.ShapeDtypeStruct((B * K_CODES, EMBED_DIM), jnp.float32),
                  jax.ShapeDtypeStruct((B, K_CODES), jnp.float32)],
        mesh=plsc.VectorSubcoreMesh(core_axis_name="c", subcore_axis_name="s",
                                    num_cores=_NC, num_subcores=_NS),
        compiler_params=pltpu.CompilerParams(use_tc_tiling_on_sc=True,
                                             needs_layout_passes=False),
        scratch_types=[
            pltpu.VMEM((_HTOK,), jnp.int32),
            pltpu.VMEM((2, _CH, _CW), jnp.float32),
            pltpu.VMEM((K_CODES, _CW), jnp.float32),
            pltpu.VMEM((K_CODES,), jnp.float32),
            pltpu.VMEM((16,), jnp.float32),
            pltpu.SemaphoreType.DMA,
            pltpu.SemaphoreType.DMA,
        ],
    )(_sc_body)


_BQ = 512
_NQ = SEQ // _BQ
_NP = NUM_HEADS // 2


def _attn_body(q_ref, cb_ref, bt_ref, v_ref, c_ref, o_ref, k2_ref):
    cvec = c_ref[0, 0, :]
    pos = cvec > 0.0
    safe = jnp.maximum(cvec, 1.0)
    bias = jnp.where(pos, jnp.log(safe), -1e30)
    inv = jnp.where(pos, 1.0 / safe, 0.0)

    @pl.when(pl.program_id(2) == 0)
    def _():
        cbf_t = lax.dot_general(cb_ref[...], bt_ref[...],
                                (((0,), (0,)), ((), ())),
                                preferred_element_type=jnp.float32)
        z64 = jnp.zeros((HEAD_DIM, K_CODES), jnp.float32)
        k2_ref[...] = jnp.concatenate(
            [jnp.concatenate([cbf_t[:HEAD_DIM], z64], axis=1),
             jnp.concatenate([z64, cbf_t[HEAD_DIM:]], axis=1)], axis=0)

    s = lax.dot_general(q_ref[...], k2_ref[...], (((1,), (0,)), ((), ())),
                        preferred_element_type=jnp.float32)
    s = s * SCALE + jnp.concatenate([bias, bias])[None, :]
    p = jnp.exp(s)
    pa = p[:, :K_CODES]
    pb = p[:, K_CODES:]
    ra = 1.0 / jnp.sum(pa, axis=1, keepdims=True)
    rb = 1.0 / jnp.sum(pb, axis=1, keepdims=True)
    wa = (pa * ra) * inv[None, :]
    wb = (pb * rb) * inv[None, :]
    na = lax.dot_general(wa, v_ref[:, :HEAD_DIM], (((1,), (0,)), ((), ())),
                         preferred_element_type=jnp.float32)
    nb = lax.dot_general(wb, v_ref[:, HEAD_DIM:], (((1,), (0,)), ((), ())),
                         preferred_element_type=jnp.float32)
    o_ref[...] = jnp.concatenate([na, nb], axis=1)


def _attn_call(q, codebook, base_t, cod_v, counts3):
    return pl.pallas_call(
        _attn_body,
        grid=(B, _NP, _NQ),
        in_specs=[
            pl.BlockSpec((_BQ, 2 * HEAD_DIM), lambda b, p, j: (b * _NQ + j, p)),
            pl.BlockSpec((2 * CODE_SIZE, 2 * HEAD_DIM), lambda b, p, j: (0, p)),
            pl.BlockSpec((2 * CODE_SIZE, K_CODES), lambda b, p, j: (0, 0)),
            pl.BlockSpec((K_CODES, 2 * HEAD_DIM), lambda b, p, j: (b, p)),
            pl.BlockSpec((1, 1, K_CODES), lambda b, p, j: (b, 0, 0)),
        ],
        out_specs=pl.BlockSpec((_BQ, 2 * HEAD_DIM), lambda b, p, j: (b * _NQ + j, p)),
        out_shape=jax.ShapeDtypeStruct((L_TOTAL, EMBED_DIM), jnp.float32),
        scratch_shapes=[pltpu.VMEM((2 * HEAD_DIM, 2 * K_CODES), jnp.float32)],
    )(q, codebook, base_t, cod_v, counts3)


def kernel(q, k, v, code_proj_w, code_proj_b, codebook, lengths, inv_lengths):
    del lengths, inv_lengths
    b_bcast = jnp.broadcast_to(code_proj_b[:, None], (CODE_SIZE, 128))
    seg = _hash_call(k, code_proj_w, b_bcast)
    zv = jnp.zeros((K_CODES, _CW), jnp.float32)
    zc = jnp.zeros((K_CODES,), jnp.float32)
    ones = jnp.ones((16,), jnp.float32)
    cod_v, cnt = _sc_scatter_fn()(v, seg, zv, zc, ones)
    counts3 = cnt.reshape(B, 1, K_CODES)
    return _attn_call(q, codebook, jnp.asarray(_BASE_CODE.T), cod_v, counts3)


# speedup vs baseline: 1.0885x; 1.0044x over previous
import functools

import jax
import jax.numpy as jnp
import numpy as np
from jax import lax
from jax.experimental import pallas as pl
from jax.experimental.pallas import tpu as pltpu
from jax.experimental.pallas import tpu_sc as plsc

EMBED_DIM = 1024
NUM_HEADS = 16
HEAD_DIM = 64
CODE_SIZE = 8
K_CODES = 256
SCALE = HEAD_DIM ** (-0.5)
B = 4
SEQ = 2048
L_TOTAL = B * SEQ

_code_base = 2 ** np.arange(CODE_SIZE - 1, -1, -1)
_bits = (np.arange(K_CODES)[:, None] // _code_base) % 2
_BASE_CODE = np.concatenate([_bits, 1 - _bits], axis=-1).astype(np.float32)

_HASH_BLK = 1024


def _hash_body(k_ref, w_ref, b_ref, seg_ref):
    blk = pl.program_id(0)
    s8 = lax.dot_general(w_ref[...], k_ref[...], (((1,), (1,)), ((), ())),
                         preferred_element_type=jnp.float32)
    s8 = s8 + b_ref[:, 0:1]
    bits = (s8 >= 0.0).astype(jnp.int32)
    j = lax.broadcasted_iota(jnp.int32, (CODE_SIZE, _HASH_BLK), 0)
    pw = jnp.left_shift(1, (CODE_SIZE - 1) - j)
    del blk
    idx = jnp.sum(bits * pw, axis=0, keepdims=True)
    seg_ref[...] = idx[None]


def _hash_call(k, w, b_bcast):
    nblk = L_TOTAL // _HASH_BLK
    seg3 = pl.pallas_call(
        _hash_body,
        grid=(nblk,),
        in_specs=[
            pl.BlockSpec((_HASH_BLK, EMBED_DIM), lambda i: (i, 0)),
            pl.BlockSpec((CODE_SIZE, EMBED_DIM), lambda i: (0, 0)),
            pl.BlockSpec((CODE_SIZE, 128), lambda i: (0, 0)),
        ],
        out_specs=pl.BlockSpec((1, 1, _HASH_BLK), lambda i: (i, 0, 0)),
        out_shape=jax.ShapeDtypeStruct((nblk, 1, _HASH_BLK), jnp.int32),
    )(k, w, b_bcast)
    return seg3.reshape(L_TOTAL)


_NC = 2
_NS = 16
_CHUNK = 64
_TOK_PER_TILE = L_TOTAL // (_NC * _NS)
_NCHUNK = _TOK_PER_TILE // _CHUNK
_SEGS = 2 * K_CODES
_ROWS_PER_TILE = _SEGS // _NS


_NCG = 8
_CW = EMBED_DIM // _NCG
_HTOK = SEQ
_CH = 128
_NCH = _HTOK // _CH


def _sc_body(v_hbm, seg_hbm, zv_hbm, zc_hbm, ones_hbm, cod_v_hbm, cnt_hbm,
             idx_all, vbuf, acc, cnt, one_v, sem0, sem1):
    c = lax.axis_index("c")
    s = lax.axis_index("s")
    cg = s % _NCG
    h = s // _NCG
    b = 2 * c + h
    row0 = b * SEQ
    col0 = cg * _CW

    pltpu.sync_copy(zv_hbm, acc)
    pltpu.sync_copy(zc_hbm, cnt)
    pltpu.sync_copy(ones_hbm, one_v)
    pltpu.sync_copy(seg_hbm.at[pl.ds(row0, _HTOK)], idx_all)
    ov = one_v[...]

    sems = (sem0, sem1)

    pltpu.async_copy(v_hbm.at[pl.ds(row0, _CH), pl.ds(col0, _CW)],
                     vbuf.at[0], sems[0])

    def _chunk_pair(jj, _):
        for bsel in range(2):
            j = jj * 2 + bsel
            nxt = ((j + 1) % _NCH) * _CH
            pltpu.async_copy(
                v_hbm.at[pl.ds(row0 + nxt, _CH), pl.ds(col0, _CW)],
                vbuf.at[1 - bsel], sems[1 - bsel])
            pltpu.make_async_copy(
                v_hbm.at[pl.ds(row0, _CH), pl.ds(col0, _CW)],
                vbuf.at[bsel], sems[bsel]).wait()
            vb = vbuf.at[bsel]

            @plsc.parallel_loop(0, _CH // 16)
            def _grp(g):
                segv = idx_all[pl.ds(j * _CH + g * 16, 16)]
                for i in range(16):
                    sr = segv[i]
                    r = g * 16 + i
                    for q in range(_CW // 16):
                        plsc.addupdate(acc.at[sr, pl.ds(q * 16, 16)],
                                       vb[r, pl.ds(q * 16, 16)])
        return 0

    lax.fori_loop(0, _NCH // 2, _chunk_pair, 0)
    pltpu.make_async_copy(v_hbm.at[pl.ds(row0, _CH), pl.ds(col0, _CW)],
                          vbuf.at[0], sems[0]).wait()

    @pl.when(cg == 0)
    def _():
        def _cgrp(g, _):
            segv = idx_all[pl.ds(g * 16, 16)]
            plsc.addupdate_scatter(cnt, [segv], ov)
            return 0

        lax.fori_loop(0, _HTOK // 16, _cgrp, 0)
        pltpu.sync_copy(cnt, cnt_hbm.at[b])

    pltpu.sync_copy(acc, cod_v_hbm.at[pl.ds(b * K_CODES, K_CODES),
                                      pl.ds(col0, _CW)])


@functools.cache
def _sc_scatter_fn():
    return functools.partial(
        pl.kernel,
        out_type=[jax.ShapeDtypeStruct((B * K_CODES, EMBED_DIM), jnp.float32),
                  jax.ShapeDtypeStruct((B, K_CODES), jnp.float32)],
        mesh=plsc.VectorSubcoreMesh(core_axis_name="c", subcore_axis_name="s",
                                    num_cores=_NC, num_subcores=_NS),
        compiler_params=pltpu.CompilerParams(use_tc_tiling_on_sc=True,
                                             needs_layout_passes=False),
        scratch_types=[
            pltpu.VMEM((_HTOK,), jnp.int32),
            pltpu.VMEM((2, _CH, _CW), jnp.float32),
            pltpu.VMEM((K_CODES, _CW), jnp.float32),
            pltpu.VMEM((K_CODES,), jnp.float32),
            pltpu.VMEM((16,), jnp.float32),
            pltpu.SemaphoreType.DMA,
            pltpu.SemaphoreType.DMA,
        ],
    )(_sc_body)


_BQ = 512
_NQ = SEQ // _BQ
_NP = NUM_HEADS // 2


def _attn_body(q_ref, cb_ref, bt_ref, v_ref, c_ref, o_ref, k2_ref):
    cvec = c_ref[0, 0, :]
    pos = cvec > 0.0
    safe = jnp.maximum(cvec, 1.0)
    bias = jnp.where(pos, jnp.log(safe), -1e30)
    inv = jnp.where(pos, 1.0 / safe, 0.0)

    @pl.when(pl.program_id(2) == 0)
    def _():
        cbf_t = lax.dot_general(cb_ref[...], bt_ref[...],
                                (((0,), (0,)), ((), ())),
                                preferred_element_type=jnp.float32)
        z64 = jnp.zeros((HEAD_DIM, K_CODES), jnp.float32)
        k2_ref[...] = jnp.concatenate(
            [jnp.concatenate([cbf_t[:HEAD_DIM], z64], axis=1),
             jnp.concatenate([z64, cbf_t[HEAD_DIM:]], axis=1)],
            axis=0).astype(jnp.bfloat16)

    s = lax.dot_general(q_ref[...].astype(jnp.bfloat16), k2_ref[...],
                        (((1,), (0,)), ((), ())),
                        preferred_element_type=jnp.float32)
    s = s * SCALE + jnp.concatenate([bias, bias])[None, :]
    p = jnp.exp(s)
    pa = p[:, :K_CODES]
    pb = p[:, K_CODES:]
    ra = 1.0 / jnp.sum(pa, axis=1, keepdims=True)
    rb = 1.0 / jnp.sum(pb, axis=1, keepdims=True)
    wa = (pa * ra) * inv[None, :]
    wb = (pb * rb) * inv[None, :]
    na = lax.dot_general(wa, v_ref[:, :HEAD_DIM], (((1,), (0,)), ((), ())),
                         preferred_element_type=jnp.float32)
    nb = lax.dot_general(wb, v_ref[:, HEAD_DIM:], (((1,), (0,)), ((), ())),
                         preferred_element_type=jnp.float32)
    o_ref[...] = jnp.concatenate([na, nb], axis=1)


def _attn_call(q, codebook, base_t, cod_v, counts3):
    return pl.pallas_call(
        _attn_body,
        grid=(B, _NP, _NQ),
        in_specs=[
            pl.BlockSpec((_BQ, 2 * HEAD_DIM), lambda b, p, j: (b * _NQ + j, p)),
            pl.BlockSpec((2 * CODE_SIZE, 2 * HEAD_DIM), lambda b, p, j: (0, p)),
            pl.BlockSpec((2 * CODE_SIZE, K_CODES), lambda b, p, j: (0, 0)),
            pl.BlockSpec((K_CODES, 2 * HEAD_DIM), lambda b, p, j: (b, p)),
            pl.BlockSpec((1, 1, K_CODES), lambda b, p, j: (b, 0, 0)),
        ],
        out_specs=pl.BlockSpec((_BQ, 2 * HEAD_DIM), lambda b, p, j: (b * _NQ + j, p)),
        out_shape=jax.ShapeDtypeStruct((L_TOTAL, EMBED_DIM), jnp.float32),
        scratch_shapes=[pltpu.VMEM((2 * HEAD_DIM, 2 * K_CODES), jnp.bfloat16)],
    )(q, codebook, base_t, cod_v, counts3)


def kernel(q, k, v, code_proj_w, code_proj_b, codebook, lengths, inv_lengths):
    del lengths, inv_lengths
    b_bcast = jnp.broadcast_to(code_proj_b[:, None], (CODE_SIZE, 128))
    seg = _hash_call(k, code_proj_w, b_bcast)
    zv = jnp.zeros((K_CODES, _CW), jnp.float32)
    zc = jnp.zeros((K_CODES,), jnp.float32)
    ones = jnp.ones((16,), jnp.float32)
    cod_v, cnt = _sc_scatter_fn()(v, seg, zv, zc, ones)
    counts3 = cnt.reshape(B, 1, K_CODES)
    return _attn_call(q, codebook, jnp.asarray(_BASE_CODE.T), cod_v, counts3)


# speedup vs baseline: 1.1536x; 1.0599x over previous
import functools

import jax
import jax.numpy as jnp
import numpy as np
from jax import lax
from jax.experimental import pallas as pl
from jax.experimental.pallas import tpu as pltpu
from jax.experimental.pallas import tpu_sc as plsc

EMBED_DIM = 1024
NUM_HEADS = 16
HEAD_DIM = 64
CODE_SIZE = 8
K_CODES = 256
SCALE = HEAD_DIM ** (-0.5)
B = 4
SEQ = 2048
L_TOTAL = B * SEQ

_code_base = 2 ** np.arange(CODE_SIZE - 1, -1, -1)
_bits = (np.arange(K_CODES)[:, None] // _code_base) % 2
_BASE_CODE = np.concatenate([_bits, 1 - _bits], axis=-1).astype(np.float32)

_HASH_BLK = 1024


def _hash_body(k_ref, w_ref, b_ref, seg_ref):
    blk = pl.program_id(0)
    s8 = lax.dot_general(w_ref[...], k_ref[...], (((1,), (1,)), ((), ())),
                         preferred_element_type=jnp.float32)
    s8 = s8 + b_ref[:, 0:1]
    bits = (s8 >= 0.0).astype(jnp.int32)
    j = lax.broadcasted_iota(jnp.int32, (CODE_SIZE, _HASH_BLK), 0)
    pw = jnp.left_shift(1, (CODE_SIZE - 1) - j)
    del blk
    idx = jnp.sum(bits * pw, axis=0, keepdims=True)
    seg_ref[...] = idx[None]


def _hash_call(k, w, b_bcast):
    nblk = L_TOTAL // _HASH_BLK
    seg3 = pl.pallas_call(
        _hash_body,
        grid=(nblk,),
        in_specs=[
            pl.BlockSpec((_HASH_BLK, EMBED_DIM), lambda i: (i, 0)),
            pl.BlockSpec((CODE_SIZE, EMBED_DIM), lambda i: (0, 0)),
            pl.BlockSpec((CODE_SIZE, 128), lambda i: (0, 0)),
        ],
        out_specs=pl.BlockSpec((1, 1, _HASH_BLK), lambda i: (i, 0, 0)),
        out_shape=jax.ShapeDtypeStruct((nblk, 1, _HASH_BLK), jnp.int32),
    )(k, w, b_bcast)
    return seg3.reshape(L_TOTAL)


_NC = 2
_NS = 16
_CHUNK = 64
_TOK_PER_TILE = L_TOTAL // (_NC * _NS)
_NCHUNK = _TOK_PER_TILE // _CHUNK
_SEGS = 2 * K_CODES
_ROWS_PER_TILE = _SEGS // _NS


_NCG = 8
_CW = EMBED_DIM // _NCG
_HTOK = SEQ
_CH = 256
_NCH = _HTOK // _CH


def _sc_body(v_hbm, seg_hbm, zv_hbm, zc_hbm, ones_hbm, cod_v_hbm, cnt_hbm,
             idx_all, vbuf, acc, cnt, one_v, sem0, sem1):
    c = lax.axis_index("c")
    s = lax.axis_index("s")
    cg = s % _NCG
    h = s // _NCG
    b = 2 * c + h
    row0 = b * SEQ
    col0 = cg * _CW

    pltpu.sync_copy(zv_hbm, acc)
    pltpu.sync_copy(zc_hbm, cnt)
    pltpu.sync_copy(ones_hbm, one_v)
    pltpu.sync_copy(seg_hbm.at[pl.ds(row0, _HTOK)], idx_all)
    ov = one_v[...]

    sems = (sem0, sem1)

    pltpu.async_copy(v_hbm.at[pl.ds(row0, _CH), pl.ds(col0, _CW)],
                     vbuf.at[0], sems[0])

    def _chunk_pair(jj, _):
        for bsel in range(2):
            j = jj * 2 + bsel
            nxt = ((j + 1) % _NCH) * _CH
            pltpu.async_copy(
                v_hbm.at[pl.ds(row0 + nxt, _CH), pl.ds(col0, _CW)],
                vbuf.at[1 - bsel], sems[1 - bsel])
            pltpu.make_async_copy(
                v_hbm.at[pl.ds(row0, _CH), pl.ds(col0, _CW)],
                vbuf.at[bsel], sems[bsel]).wait()
            vb = vbuf.at[bsel]

            @plsc.parallel_loop(0, _CH // 16)
            def _grp(g):
                segv = idx_all[pl.ds(j * _CH + g * 16, 16)]
                for i in range(16):
                    sr = segv[i]
                    r = g * 16 + i
                    for q in range(_CW // 16):
                        plsc.addupdate(acc.at[sr, pl.ds(q * 16, 16)],
                                       vb[r, pl.ds(q * 16, 16)])
        return 0

    lax.fori_loop(0, _NCH // 2, _chunk_pair, 0)
    pltpu.make_async_copy(v_hbm.at[pl.ds(row0, _CH), pl.ds(col0, _CW)],
                          vbuf.at[0], sems[0]).wait()

    @pl.when(cg == 0)
    def _():
        def _cgrp(g, _):
            segv = idx_all[pl.ds(g * 16, 16)]
            plsc.addupdate_scatter(cnt, [segv], ov)
            return 0

        lax.fori_loop(0, _HTOK // 16, _cgrp, 0)
        pltpu.sync_copy(cnt, cnt_hbm.at[b])

    pltpu.sync_copy(acc, cod_v_hbm.at[pl.ds(b * K_CODES, K_CODES),
                                      pl.ds(col0, _CW)])


@functools.cache
def _sc_scatter_fn():
    return functools.partial(
        pl.kernel,
        out_type=[jax.ShapeDtypeStruct((B * K_CODES, EMBED_DIM), jnp.float32),
                  jax.ShapeDtypeStruct((B, K_CODES), jnp.float32)],
        mesh=plsc.VectorSubcoreMesh(core_axis_name="c", subcore_axis_name="s",
                                    num_cores=_NC, num_subcores=_NS),
        compiler_params=pltpu.CompilerParams(use_tc_tiling_on_sc=True,
                                             needs_layout_passes=False),
        scratch_types=[
            pltpu.VMEM((_HTOK,), jnp.int32),
            pltpu.VMEM((2, _CH, _CW), jnp.float32),
            pltpu.VMEM((K_CODES, _CW), jnp.float32),
            pltpu.VMEM((K_CODES,), jnp.float32),
            pltpu.VMEM((16,), jnp.float32),
            pltpu.SemaphoreType.DMA,
            pltpu.SemaphoreType.DMA,
        ],
    )(_sc_body)


_BQ = 512
_NQ = SEQ // _BQ
_NP = NUM_HEADS // 2


def _attn_body(q_ref, cb_ref, bt_ref, v_ref, c_ref, o_ref, k2_ref):
    cvec = c_ref[0, 0, :]
    pos = cvec > 0.0
    safe = jnp.maximum(cvec, 1.0)
    bias = jnp.where(pos, jnp.log(safe), -1e30)
    inv = jnp.where(pos, 1.0 / safe, 0.0)

    @pl.when(pl.program_id(2) == 0)
    def _():
        cbf_t = lax.dot_general(cb_ref[...], bt_ref[...],
                                (((0,), (0,)), ((), ())),
                                preferred_element_type=jnp.float32)
        z64 = jnp.zeros((HEAD_DIM, K_CODES), jnp.float32)
        k2_ref[...] = jnp.concatenate(
            [jnp.concatenate([cbf_t[:HEAD_DIM], z64], axis=1),
             jnp.concatenate([z64, cbf_t[HEAD_DIM:]], axis=1)],
            axis=0).astype(jnp.bfloat16)

    s = lax.dot_general(q_ref[...].astype(jnp.bfloat16), k2_ref[...],
                        (((1,), (0,)), ((), ())),
                        preferred_element_type=jnp.float32)
    s = s * SCALE + jnp.concatenate([bias, bias])[None, :]
    p = jnp.exp(s)
    pa = p[:, :K_CODES]
    pb = p[:, K_CODES:]
    ra = 1.0 / jnp.sum(pa, axis=1, keepdims=True)
    rb = 1.0 / jnp.sum(pb, axis=1, keepdims=True)
    wa = (pa * ra) * inv[None, :]
    wb = (pb * rb) * inv[None, :]
    na = lax.dot_general(wa, v_ref[:, :HEAD_DIM], (((1,), (0,)), ((), ())),
                         preferred_element_type=jnp.float32)
    nb = lax.dot_general(wb, v_ref[:, HEAD_DIM:], (((1,), (0,)), ((), ())),
                         preferred_element_type=jnp.float32)
    o_ref[...] = jnp.concatenate([na, nb], axis=1)


def _attn_call(q, codebook, base_t, cod_v, counts3):
    return pl.pallas_call(
        _attn_body,
        grid=(B, _NP, _NQ),
        in_specs=[
            pl.BlockSpec((_BQ, 2 * HEAD_DIM), lambda b, p, j: (b * _NQ + j, p)),
            pl.BlockSpec((2 * CODE_SIZE, 2 * HEAD_DIM), lambda b, p, j: (0, p)),
            pl.BlockSpec((2 * CODE_SIZE, K_CODES), lambda b, p, j: (0, 0)),
            pl.BlockSpec((K_CODES, 2 * HEAD_DIM), lambda b, p, j: (b, p)),
            pl.BlockSpec((1, 1, K_CODES), lambda b, p, j: (b, 0, 0)),
        ],
        out_specs=pl.BlockSpec((_BQ, 2 * HEAD_DIM), lambda b, p, j: (b * _NQ + j, p)),
        out_shape=jax.ShapeDtypeStruct((L_TOTAL, EMBED_DIM), jnp.float32),
        scratch_shapes=[pltpu.VMEM((2 * HEAD_DIM, 2 * K_CODES), jnp.bfloat16)],
    )(q, codebook, base_t, cod_v, counts3)


def kernel(q, k, v, code_proj_w, code_proj_b, codebook, lengths, inv_lengths):
    del lengths, inv_lengths
    b_bcast = jnp.broadcast_to(code_proj_b[:, None], (CODE_SIZE, 128))
    seg = _hash_call(k, code_proj_w, b_bcast)
    zv = jnp.zeros((K_CODES, _CW), jnp.float32)
    zc = jnp.zeros((K_CODES,), jnp.float32)
    ones = jnp.ones((16,), jnp.float32)
    cod_v, cnt = _sc_scatter_fn()(v, seg, zv, zc, ones)
    counts3 = cnt.reshape(B, 1, K_CODES)
    return _attn_call(q, codebook, jnp.asarray(_BASE_CODE.T), cod_v, counts3)
